# baseline (speedup 1.0000x reference)
import functools

import jax
import jax.numpy as jnp
from jax import lax
from jax.experimental import pallas as pl
from jax.experimental.pallas import tpu as pltpu
from jax.experimental.pallas import tpu_sc as plsc

VOCAB = 100000
EMB = 128
CS = 64
CTX = 50
NEG = 10

NW = 32
ROWS = 1024
RPW = ROWS // NW
EROWS = 64


def _sc_gather(idx_all, idx_e, prior_mus, prior_sigmas, E):
    mesh = plsc.VectorSubcoreMesh(core_axis_name="c", subcore_axis_name="s")

    @functools.partial(
        pl.kernel,
        out_type=(
            jax.ShapeDtypeStruct((ROWS, CS), jnp.float32),
            jax.ShapeDtypeStruct((ROWS, CS), jnp.float32),
            jax.ShapeDtypeStruct((EROWS, EMB), jnp.float32),
        ),
        mesh=mesh,
        compiler_params=pltpu.CompilerParams(use_tc_tiling_on_sc=False),
        scratch_types=(
            pltpu.VMEM((RPW,), jnp.int32),
            pltpu.VMEM((RPW, CS), jnp.float32),
            pltpu.VMEM((RPW, CS), jnp.float32),
            pltpu.VMEM((8,), jnp.int32),
            pltpu.VMEM((8, EMB), jnp.float32),
            pltpu.SemaphoreType.DMA,
            pltpu.SemaphoreType.DMA,
            pltpu.SemaphoreType.DMA,
        ),
    )
    def k(idx_hbm, idx_e_hbm, pm_hbm, ps_hbm, e_hbm, out_pm, out_ps, out_e,
          idx_v, rows_pm, rows_ps, idx_e_v, rows_e, sem0, sem1, sem2):
        wid = lax.axis_index("s") * 2 + lax.axis_index("c")
        base = wid * RPW
        pltpu.sync_copy(idx_hbm.at[pl.ds(base, RPW)], idx_v)
        cp0 = pltpu.async_copy(pm_hbm.at[idx_v], rows_pm, sem0)
        cp1 = pltpu.async_copy(ps_hbm.at[idx_v], rows_ps, sem1)

        @pl.when(wid < 8)
        def _():
            ebase = wid * 8
            pltpu.sync_copy(idx_e_hbm.at[pl.ds(ebase, 8)], idx_e_v)
            pltpu.async_copy(e_hbm.at[idx_e_v], rows_e, sem2).wait()
            pltpu.sync_copy(rows_e, out_e.at[pl.ds(ebase, 8)])

        cp0.wait()
        pltpu.sync_copy(rows_pm, out_pm.at[pl.ds(base, RPW)])
        cp1.wait()
        pltpu.sync_copy(rows_ps, out_ps.at[pl.ds(base, RPW)])

    return k(idx_all, idx_e, prior_mus, prior_sigmas, E)


def _tc_body(pm_ref, ps_ref, e_ref, mw_ref, mb_ref, uw_ref, ub_ref,
             ww_ref, wb_ref, out_ref):
    f32 = jnp.float32
    hi = jax.lax.Precision.HIGHEST

    ex = e_ref[0:8, :]
    ec = e_ref[8:64, :]
    rw = jax.nn.relu(
        lax.dot_general(ex, mw_ref[...], (((1,), (1,)), ((), ())),
                        precision=hi, preferred_element_type=f32)
        + mb_ref[...])
    rc = jax.nn.relu(
        lax.dot_general(ec, mw_ref[...], (((1,), (1,)), ((), ())),
                        precision=hi, preferred_element_type=f32)
        + mb_ref[...])
    crow = lax.broadcasted_iota(jnp.int32, (56, CS), 0)
    rc = jnp.where(crow < CTX, rc, 0.0)
    h1 = CTX * rw[0:1, :]
    h2 = jnp.sum(rc, axis=0, keepdims=True)
    h = jnp.concatenate([h1, h2], axis=1)

    mu = lax.dot_general(h, uw_ref[...], (((1,), (1,)), ((), ())),
                         precision=hi, preferred_element_type=f32) + ub_ref[...]
    z = lax.dot_general(h, ww_ref[...], (((1,), (1,)), ((), ())),
                        precision=hi, preferred_element_type=f32) + wb_ref[...]
    post_var = jax.nn.softplus(z)
    log_post_var = jnp.sum(jnp.log(post_var))

    pm = pm_ref[...]
    ps = ps_ref[...]
    v = ps * ps
    a = jnp.sum((post_var + (pm - mu) ** 2) / v, axis=1, keepdims=True)
    b = 2.0 * jnp.sum(jnp.log(ps), axis=1, keepdims=True)
    kl = 0.5 * (a + b - CS - log_post_var)

    kl_x = kl[0:1, 0:1]
    kl_pos = kl[64:128, :]
    kl_neg = kl[512:1024, :]

    irow = lax.broadcasted_iota(jnp.int32, (512, 64), 0)
    icol = lax.broadcasted_iota(jnp.int32, (512, 64), 1)
    sel = jnp.where(irow // NEG == icol, 1.0, 0.0).astype(f32)
    pos_for_neg = lax.dot_general(sel, kl_pos, (((1,), (0,)), ((), ())),
                                  precision=hi, preferred_element_type=f32)
    hinge = jnp.maximum(kl_neg - pos_for_neg + 1.0, 0.0)
    nrow = lax.broadcasted_iota(jnp.int32, (512, 1), 0)
    hinge = jnp.where(nrow < CTX * NEG, hinge, 0.0)
    likelihood = jnp.sum(hinge, keepdims=True)
    out_ref[...] = likelihood - kl_x


def kernel(x, context, neg_samples, E, M_w, M_b, U_w, U_b, W_w, W_b,
           prior_mus, prior_sigmas):
    zi = jnp.zeros((), jnp.int32)
    x = x.astype(jnp.int32)
    context = context.astype(jnp.int32)
    negf = neg_samples.reshape(-1).astype(jnp.int32)
    idx_all = jnp.concatenate([
        x, jnp.full((63,), zi), context, jnp.full((398,), zi),
        negf, jnp.full((12,), zi)])
    idx_e = jnp.concatenate([x, jnp.full((7,), zi), context,
                             jnp.full((6,), zi)])

    pm_g, ps_g, e_g = _sc_gather(idx_all, idx_e, prior_mus, prior_sigmas, E)

    out = pl.pallas_call(
        _tc_body,
        out_shape=jax.ShapeDtypeStruct((1, 1), jnp.float32),
    )(pm_g, ps_g, e_g,
      M_w, M_b.reshape(1, CS), U_w, U_b.reshape(1, CS),
      W_w, W_b.reshape(1, CS))
    return out.reshape((1,))


# speedup vs baseline: 1.3332x; 1.3332x over previous
import functools

import jax
import jax.numpy as jnp
from jax import lax
from jax.experimental import pallas as pl
from jax.experimental.pallas import tpu as pltpu
from jax.experimental.pallas import tpu_sc as plsc

VOCAB = 100000
EMB = 128
CS = 64
CTX = 50
NEG = 10

ROWS = 576
K = 32
STEPS = ROWS // K
EROWS = 64


def _sc_gather_e(idx_e, E):
    mesh = plsc.VectorSubcoreMesh(core_axis_name="c", subcore_axis_name="s")

    @functools.partial(
        pl.kernel,
        out_type=jax.ShapeDtypeStruct((EROWS, EMB), jnp.float32),
        mesh=mesh,
        scratch_types=(
            pltpu.VMEM((8,), jnp.int32),
            pltpu.VMEM((8, EMB), jnp.float32),
            pltpu.SemaphoreType.DMA,
        ),
    )
    def k(idx_e_hbm, e_hbm, out_e, idx_e_v, rows_e, sem):
        wid = lax.axis_index("s") * 2 + lax.axis_index("c")

        @pl.when(wid < 8)
        def _():
            ebase = wid * 8
            pltpu.sync_copy(idx_e_hbm.at[pl.ds(ebase, 8)], idx_e_v)
            pltpu.async_copy(e_hbm.at[idx_e_v], rows_e, sem).wait()
            pltpu.sync_copy(rows_e, out_e.at[pl.ds(ebase, 8)])

    return k(idx_e, E)


def _feat_body(idx_sref, subs_ref, *refs):
    pm_blks = refs[0:K]
    ps_blks = refs[K:2 * K]
    r_out, q_out, tb_out = refs[2 * K:2 * K + 3]
    f32 = jnp.float32

    sub = subs_ref[...][:, 0:1]
    pm = jnp.concatenate([b[...] for b in pm_blks], axis=0)
    ps = jnp.concatenate([b[...] for b in ps_blks], axis=0)
    srow = lax.broadcasted_iota(jnp.int32, (K, 8), 1)
    onehot = (srow == sub).astype(f32)[:, :, None]
    m = jnp.sum(pm * onehot, axis=1)
    s = jnp.sum(ps * onehot, axis=1)

    v = s * s
    r = 1.0 / v
    q = m * r
    t = jnp.sum(m * q, axis=1, keepdims=True)
    b = 2.0 * jnp.sum(jnp.log(s), axis=1, keepdims=True)
    lane = lax.broadcasted_iota(jnp.int32, (K, CS), 1)
    tb = jnp.where(lane == 0, t, 0.0) + jnp.where(lane == 1, b, 0.0)
    r_out[...] = r
    q_out[...] = q
    tb_out[...] = tb


def _prior_features(idx_all, subs128, pm3, ps3):
    def mk_spec(k):
        return pl.BlockSpec(
            (1, 8, CS), lambda i, idx, _k=k: (idx[K * i + _k] // 8, 0, 0))

    grid_spec = pltpu.PrefetchScalarGridSpec(
        num_scalar_prefetch=1,
        grid=(STEPS,),
        in_specs=[pl.BlockSpec((K, 128), lambda i, idx: (i, 0))]
        + [mk_spec(k) for k in range(K)] * 2,
        out_specs=[pl.BlockSpec((K, CS), lambda i, idx: (i, 0))] * 3,
    )
    return pl.pallas_call(
        _feat_body,
        grid_spec=grid_spec,
        out_shape=[jax.ShapeDtypeStruct((ROWS, CS), jnp.float32)] * 3,
    )(idx_all, subs128, *([pm3] * K), *([ps3] * K))


def _combine_body(e_ref, r_ref, q_ref, tb_ref, mw_ref, mb_ref, uw_ref,
                  ub_ref, ww_ref, wb_ref, out_ref):
    f32 = jnp.float32
    hi = jax.lax.Precision.HIGHEST

    ex = e_ref[0:8, :]
    ec = e_ref[8:64, :]
    rw = jax.nn.relu(
        lax.dot_general(ex, mw_ref[...], (((1,), (1,)), ((), ())),
                        precision=hi, preferred_element_type=f32)
        + mb_ref[...])
    rc = jax.nn.relu(
        lax.dot_general(ec, mw_ref[...], (((1,), (1,)), ((), ())),
                        precision=hi, preferred_element_type=f32)
        + mb_ref[...])
    crow = lax.broadcasted_iota(jnp.int32, (56, CS), 0)
    rc = jnp.where(crow < CTX, rc, 0.0)
    h1 = CTX * rw[0:1, :]
    h2 = jnp.sum(rc, axis=0, keepdims=True)
    h = jnp.concatenate([h1, h2], axis=1)

    mu = lax.dot_general(h, uw_ref[...], (((1,), (1,)), ((), ())),
                         precision=hi, preferred_element_type=f32) + ub_ref[...]
    z = lax.dot_general(h, ww_ref[...], (((1,), (1,)), ((), ())),
                        precision=hi, preferred_element_type=f32) + wb_ref[...]
    post_var = jax.nn.softplus(z)
    log_post_var = jnp.sum(jnp.log(post_var))

    pvmu = post_var + mu * mu
    a1 = lax.dot_general(r_ref[...], pvmu, (((1,), (1,)), ((), ())),
                         precision=hi, preferred_element_type=f32)
    a2 = lax.dot_general(q_ref[...], mu, (((1,), (1,)), ((), ())),
                         precision=hi, preferred_element_type=f32)
    t = tb_ref[:, 0:1]
    b = tb_ref[:, 1:2]
    a = a1 - 2.0 * a2 + t
    kl = 0.5 * (a + b - CS - log_post_var)

    kl_x = kl[0:1, 0:1]
    kl_pos = kl[8:64, :]
    kl_neg = kl[64:576, :]

    irow = lax.broadcasted_iota(jnp.int32, (512, 56), 0)
    icol = lax.broadcasted_iota(jnp.int32, (512, 56), 1)
    sel = jnp.where(irow // NEG == icol, 1.0, 0.0).astype(f32)
    pos_for_neg = lax.dot_general(sel, kl_pos, (((1,), (0,)), ((), ())),
                                  precision=hi, preferred_element_type=f32)
    hinge = jnp.maximum(kl_neg - pos_for_neg + 1.0, 0.0)
    nrow = lax.broadcasted_iota(jnp.int32, (512, 1), 0)
    hinge = jnp.where(nrow < CTX * NEG, hinge, 0.0)
    likelihood = jnp.sum(hinge, keepdims=True)
    out_ref[...] = likelihood - kl_x


def kernel(x, context, neg_samples, E, M_w, M_b, U_w, U_b, W_w, W_b,
           prior_mus, prior_sigmas):
    zi = jnp.zeros((), jnp.int32)
    x = x.astype(jnp.int32)
    context = context.astype(jnp.int32)
    negf = neg_samples.reshape(-1).astype(jnp.int32)
    idx_all = jnp.concatenate([
        x, jnp.full((7,), zi), context, jnp.full((6,), zi),
        negf, jnp.full((12,), zi)])
    idx_e = idx_all[:EROWS]
    subs128 = jnp.broadcast_to((idx_all % 8)[:, None], (ROWS, 128))

    pm3 = prior_mus.reshape(VOCAB // 8, 8, CS)
    ps3 = prior_sigmas.reshape(VOCAB // 8, 8, CS)

    e_g = _sc_gather_e(idx_e, E)
    r_f, q_f, tb_f = _prior_features(idx_all, subs128, pm3, ps3)

    out = pl.pallas_call(
        _combine_body,
        out_shape=jax.ShapeDtypeStruct((1, 1), jnp.float32),
    )(e_g, r_f, q_f, tb_f,
      M_w, M_b.reshape(1, CS), U_w, U_b.reshape(1, CS),
      W_w, W_b.reshape(1, CS))
    return out.reshape((1,))
